# baseline (speedup 1.0000x reference)
import jax
import jax.numpy as jnp
from jax import lax
from jax.experimental import pallas as pl
from jax.experimental.pallas import tpu as pltpu
from jax.experimental.pallas import tpu_sc as plsc

N = 10000
E = 160000
IN_CH = 256
OUT_CH = 64
HEADS = 4
HC = HEADS * OUT_CH
NEG_SLOPE = 0.2

BN = 400
NROW = N // BN

NTILES = 32
CHB = 128
EPT_B = 5120
E_PAD = NTILES * EPT_B
NCHB = EPT_B // CHB
NPT = N // 16
FB = 128
BLK = 1280
PEND = 1424


def _proj_body(x_ref, w_ref, acat_ref, h0_ref, h1_ref, aseg_ref, m_ref):
    i = pl.program_id(0)
    j = pl.program_id(1)
    h = jnp.dot(x_ref[...], w_ref[...], preferred_element_type=jnp.float32)
    part = jnp.dot(h, acat_ref[0], preferred_element_type=jnp.float32)

    @pl.when(j == 0)
    def _():
        h0_ref[...] = h
        aseg_ref[...] = part

    @pl.when(j == 1)
    def _():
        h1_ref[...] = h
        tot = aseg_ref[...] + part
        aseg_ref[...] = tot
        rowmax = jnp.max(tot, axis=0, keepdims=True)
        m_ref[...] = jnp.where(i == 0, rowmax, jnp.maximum(m_ref[...], rowmax))


_phase_a = pl.pallas_call(
    _proj_body,
    grid=(NROW, 2),
    in_specs=[
        pl.BlockSpec((BN, IN_CH), lambda i, j: (i, 0)),
        pl.BlockSpec((IN_CH, 128), lambda i, j: (0, j)),
        pl.BlockSpec((1, 128, 2 * HEADS), lambda i, j: (j, 0, 0)),
    ],
    out_specs=[
        pl.BlockSpec((BN, 128), lambda i, j: (i, 0)),
        pl.BlockSpec((BN, 128), lambda i, j: (i, 0)),
        pl.BlockSpec((BN, 2 * HEADS), lambda i, j: (i, 0)),
        pl.BlockSpec((1, 2 * HEADS), lambda i, j: (0, 0)),
    ],
    out_shape=[
        jax.ShapeDtypeStruct((N, 128), jnp.float32),
        jax.ShapeDtypeStruct((N, 128), jnp.float32),
        jax.ShapeDtypeStruct((N, 2 * HEADS), jnp.float32),
        jax.ShapeDtypeStruct((1, 2 * HEADS), jnp.float32),
    ],
)

_mesh = plsc.VectorSubcoreMesh(
    core_axis_name="c", subcore_axis_name="s", num_cores=2, num_subcores=16
)
_sc_params = pltpu.CompilerParams(
    needs_layout_passes=False, use_tc_tiling_on_sc=False
)


def _phase_b_body(aseg_hbm, src_hbm, dst_hbm, m16_hbm, z8_hbm, rec_hbm,
                  sp32_hbm, adb, asb, rec_c, srcc, dstc, m_v, s_loc, sem,
                  sem2):
    c = lax.axis_index("c")
    s = lax.axis_index("s")
    wid = s * 2 + c
    e0 = wid * EPT_B

    pltpu.sync_copy(m16_hbm, m_v)
    pltpu.sync_copy(z8_hbm, s_loc)

    iota = lax.iota(jnp.int32, 16)
    mh = [m_v[pl.ds(h * 16, 16)] for h in range(HEADS)]

    def chunk(ch, carry):
        eb = e0 + ch * CHB
        cpa = pltpu.async_copy(src_hbm.at[pl.ds(eb, CHB)], srcc, sem)
        cpb = pltpu.async_copy(dst_hbm.at[pl.ds(eb, CHB)], dstc, sem2)
        cpa.wait()
        cpb.wait()
        cpa = pltpu.async_copy(aseg_hbm.at[dstc], adb, sem)
        cpb = pltpu.async_copy(aseg_hbm.at[srcc], asb, sem2)
        cpa.wait()
        cpb.wait()
        for g in range(CHB // 16):
            rows = g * 16 + iota
            msk = (eb + g * 16 + iota) < E
            d = dstc[pl.ds(g * 16, 16)]
            sv = srcc[pl.ds(g * 16, 16)]
            plsc.store_scatter(rec_c, [rows, jnp.full((16,), 0, jnp.int32)],
                               plsc.bitcast(sv, jnp.float32))
            plsc.store_scatter(rec_c, [rows, jnp.full((16,), 1, jnp.int32)],
                               plsc.bitcast(d, jnp.float32))
            for h in range(HEADS):
                hd = jnp.full((16,), h, jnp.int32)
                hs = jnp.full((16,), HEADS + h, jnp.int32)
                ad = plsc.load_gather(adb, [rows, hd])
                aj = plsc.load_gather(asb, [rows, hs])
                a = ad + aj
                a = jnp.where(a >= 0.0, a, jnp.float32(NEG_SLOPE) * a)
                p = jnp.exp(a - mh[h])
                p = jnp.where(msk, p, 0.0)
                plsc.store_scatter(rec_c,
                                   [rows, jnp.full((16,), 2 + h, jnp.int32)],
                                   p)
                plsc.addupdate_scatter(s_loc, [d, hd], p)
        pltpu.sync_copy(rec_c, rec_hbm.at[pl.ds(eb, CHB)])
        return carry

    lax.fori_loop(0, NCHB, chunk, 0)

    for s2 in range(16):
        pltpu.sync_copy(s_loc.at[pl.ds(NPT * s2, NPT)], sp32_hbm.at[wid, s2])


_phase_b = pl.kernel(
    _phase_b_body,
    out_type=(
        jax.ShapeDtypeStruct((E_PAD, 8), jnp.float32),
        jax.ShapeDtypeStruct((NTILES, 16, NPT, 8), jnp.float32),
    ),
    mesh=_mesh,
    compiler_params=_sc_params,
    scratch_types=[
        pltpu.VMEM((CHB, 2 * HEADS), jnp.float32),
        pltpu.VMEM((CHB, 2 * HEADS), jnp.float32),
        pltpu.VMEM((CHB, 8), jnp.float32),
        pltpu.VMEM((CHB,), jnp.int32),
        pltpu.VMEM((CHB,), jnp.int32),
        pltpu.VMEM((4 * 16,), jnp.float32),
        pltpu.VMEM((N, 8), jnp.float32),
        pltpu.SemaphoreType.DMA,
        pltpu.SemaphoreType.DMA,
    ],
)


def _phase_c_body(h0_hbm, h1_hbm, rec_hbm, sd_hbm, sp32_hbm, bias_hbm,
                  z128_hbm, out_hbm,
                  acc, rec_sel, h_sel, srcsel, fbuf, pend, pendsrc, sdblk,
                  stmp, rsum, bias_v, sem, sem2):
    c = lax.axis_index("c")
    s = lax.axis_index("s")
    n0 = s * NPT

    pltpu.sync_copy(z128_hbm.at[pl.ds(n0, NPT)], acc)
    iota = lax.iota(jnp.int32, 16)

    def zf(g, carry):
        pend[pl.ds(g * 16, 16)] = jnp.zeros((16,), jnp.int32)
        return carry

    lax.fori_loop(0, PEND // 16, zf, 0)

    def main(h_hbm, cc):
        def flush(cnt, nproc):
            base = cnt - nproc if isinstance(nproc, int) else 0
            for g2 in range(FB // 16):
                ids = plsc.load_gather(pend, [base + g2 * 16 + iota])
                fbuf[pl.ds(g2 * 16, 16)] = ids
                sv = plsc.load_gather(pendsrc, [base + g2 * 16 + iota])
                srcsel[pl.ds(g2 * 16, 16)] = sv
            cp1 = pltpu.async_copy(rec_hbm.at[fbuf], rec_sel, sem)
            cp2 = pltpu.async_copy(h_hbm.at[srcsel], h_sel, sem2)
            cp1.wait()
            cp2.wait()

            def per_edge(e, carry2):
                er = jnp.full((16,), e, jnp.int32)
                dsp = plsc.bitcast(
                    plsc.load_gather(rec_sel,
                                     [er, jnp.full((16,), 1, jnp.int32)]),
                    jnp.int32)
                rel = dsp - n0
                w0 = plsc.load_gather(
                    rec_sel, [er, jnp.full((16,), 2 + 2 * cc, jnp.int32)])
                w1 = plsc.load_gather(
                    rec_sel, [er, jnp.full((16,), 3 + 2 * cc, jnp.int32)])
                for j8 in range(8):
                    hv = h_sel[e, pl.ds(j8 * 16, 16)]
                    w = w0 if j8 < 4 else w1
                    plsc.addupdate_scatter(acc, [rel, j8 * 16 + iota], hv * w)
                return carry2

            lax.fori_loop(0, nproc, per_edge, 0)

        def block(b, cnt):
            pltpu.sync_copy(sd_hbm.at[pl.ds(b * BLK, BLK)], sdblk)
            for g in range(BLK // 16):
                grows = g * 16 + iota
                d = plsc.load_gather(sdblk,
                                     [grows, jnp.full((16,), 1, jnp.int32)])
                sv = plsc.load_gather(sdblk,
                                      [grows, jnp.full((16,), 0, jnp.int32)])
                msk = jnp.logical_and(d >= n0, d < n0 + NPT)
                eid = b * BLK + g * 16 + iota
                cum = plsc.cumsum(msk.astype(jnp.int32))
                pos = cnt + cum - 1
                plsc.store_scatter(pend, [pos], eid, mask=msk)
                plsc.store_scatter(pendsrc, [pos], sv, mask=msk)
                cnt = cnt + cum[15]

            def wbody(cnt2):
                flush(cnt2, FB)
                return cnt2 - FB

            cnt = lax.while_loop(lambda c2: c2 >= FB, wbody, cnt)
            return cnt

        cnt = lax.fori_loop(0, E // BLK, block, jnp.int32(0))
        flush(cnt, cnt)

    @pl.when(c == 0)
    def _():
        main(h0_hbm, 0)

    @pl.when(c == 1)
    def _():
        main(h1_hbm, 1)

    def writeout(cc):
        pltpu.sync_copy(bias_hbm.at[pl.ds(cc * 128, 128)], bias_v)

        def zr(g, carry):
            rsum[pl.ds(g * 16, 16)] = jnp.zeros((16,), jnp.float32)
            return carry

        lax.fori_loop(0, (NPT * 8 + 15) // 16, zr, 0)
        for t in range(NTILES):
            pltpu.sync_copy(sp32_hbm.at[t, s], stmp.at[pl.ds(0, NPT)])

            def racc(g, carry):
                rowv = g * 2 + jnp.right_shift(iota, 3)
                colv = jnp.bitwise_and(iota, 7)
                v = plsc.load_gather(stmp, [rowv, colv])
                rsum[pl.ds(g * 16, 16)] = rsum[pl.ds(g * 16, 16)] + v
                return carry

            lax.fori_loop(0, (NPT * 8 + 15) // 16, racc, 0)

        def rinv(g, carry):
            v = rsum[pl.ds(g * 16, 16)]
            rsum[pl.ds(g * 16, 16)] = 1.0 / (v + 1e-16)
            return carry

        lax.fori_loop(0, (NPT * 8 + 15) // 16, rinv, 0)

        def row(e, carry):
            rv0 = plsc.load_gather(
                rsum, [jnp.full((16,), 0, jnp.int32) + e * 8 + 2 * cc])
            rv1 = plsc.load_gather(
                rsum, [jnp.full((16,), 0, jnp.int32) + e * 8 + 2 * cc + 1])
            for j8 in range(8):
                hv = acc[e, pl.ds(j8 * 16, 16)]
                rv = rv0 if j8 < 4 else rv1
                acc[e, pl.ds(j8 * 16, 16)] = hv * rv + bias_v[pl.ds(j8 * 16, 16)]
            return carry

        lax.fori_loop(0, NPT, row, 0)
        pltpu.sync_copy(acc, out_hbm.at[cc, pl.ds(n0, NPT)])

    @pl.when(c == 0)
    def _():
        writeout(0)

    @pl.when(c == 1)
    def _():
        writeout(1)


_phase_c = pl.kernel(
    _phase_c_body,
    out_type=jax.ShapeDtypeStruct((2, N, 128), jnp.float32),
    mesh=_mesh,
    compiler_params=_sc_params,
    scratch_types=[
        pltpu.VMEM((NPT, 128), jnp.float32),
        pltpu.VMEM((FB, 8), jnp.float32),
        pltpu.VMEM((FB, 128), jnp.float32),
        pltpu.VMEM((FB,), jnp.int32),
        pltpu.VMEM((FB,), jnp.int32),
        pltpu.VMEM((PEND,), jnp.int32),
        pltpu.VMEM((PEND,), jnp.int32),
        pltpu.VMEM((BLK, 2), jnp.int32),
        pltpu.VMEM((NPT + 1, 8), jnp.float32),
        pltpu.VMEM(((NPT * 8 + 15) // 16 * 16,), jnp.float32),
        pltpu.VMEM((128,), jnp.float32),
        pltpu.SemaphoreType.DMA,
        pltpu.SemaphoreType.DMA,
    ],
)


def kernel(x, edge_index, mask, weight, att, bias):
    src = edge_index[0].astype(jnp.int32)
    dst = edge_index[1].astype(jnp.int32)
    x = x.astype(jnp.float32)
    weight = weight.astype(jnp.float32)
    att = att.astype(jnp.float32)
    bias = bias.astype(jnp.float32)

    att1 = att[0]
    attd = att1[:, :OUT_CH]
    atts = att1[:, OUT_CH:]
    eye = jnp.eye(HEADS, dtype=jnp.float32)
    a_d = (eye[:, None, :] * attd[:, :, None]).reshape(HC, HEADS)
    a_s = (eye[:, None, :] * atts[:, :, None]).reshape(HC, HEADS)
    acat = jnp.concatenate([a_d, a_s], axis=1).reshape(2, 128, 2 * HEADS)

    h0, h1, aseg, m = _phase_a(x, weight, acat)

    m4 = m[0, :HEADS] + m[0, HEADS:]
    ml = jnp.where(m4 >= 0.0, m4, NEG_SLOPE * m4)
    m16 = jnp.repeat(ml, 16)

    srcp = jnp.concatenate([src, jnp.zeros((E_PAD - E,), jnp.int32)])
    dstp = jnp.concatenate([dst, jnp.zeros((E_PAD - E,), jnp.int32)])
    z8 = jnp.zeros((N, 8), jnp.float32)
    rec, sp32 = _phase_b(aseg, srcp, dstp, m16, z8)

    z128 = jnp.zeros((N, 128), jnp.float32)
    sd = jnp.stack([src, dst], axis=1)
    outp = _phase_c(h0, h1, rec, sd, sp32, bias, z128)
    return jnp.moveaxis(outp, 0, 1).reshape(N, HC)


# speedup vs baseline: 1.1660x; 1.1660x over previous
import jax
import jax.numpy as jnp
from jax import lax
from jax.experimental import pallas as pl
from jax.experimental.pallas import tpu as pltpu
from jax.experimental.pallas import tpu_sc as plsc

N = 10000
E = 160000
IN_CH = 256
OUT_CH = 64
HEADS = 4
HC = HEADS * OUT_CH
NEG_SLOPE = 0.2

BN = 400
NROW = N // BN

NTILES = 32
CHB = 128
EPT_B = 5120
E_PAD = NTILES * EPT_B
NCHB = EPT_B // CHB
NPT = N // 16
FB = 256
BLK = 1280
PEND = 1552


def _proj_body(x_ref, w_ref, acat_ref, h0_ref, h1_ref, aseg_ref, m_ref):
    i = pl.program_id(0)
    j = pl.program_id(1)
    h = jnp.dot(x_ref[...], w_ref[...], preferred_element_type=jnp.float32)
    part = jnp.dot(h, acat_ref[0], preferred_element_type=jnp.float32)

    @pl.when(j == 0)
    def _():
        h0_ref[...] = h
        aseg_ref[...] = part

    @pl.when(j == 1)
    def _():
        h1_ref[...] = h
        tot = aseg_ref[...] + part
        aseg_ref[...] = tot
        rowmax = jnp.max(tot, axis=0, keepdims=True)
        m_ref[...] = jnp.where(i == 0, rowmax, jnp.maximum(m_ref[...], rowmax))


_phase_a = pl.pallas_call(
    _proj_body,
    grid=(NROW, 2),
    in_specs=[
        pl.BlockSpec((BN, IN_CH), lambda i, j: (i, 0)),
        pl.BlockSpec((IN_CH, 128), lambda i, j: (0, j)),
        pl.BlockSpec((1, 128, 2 * HEADS), lambda i, j: (j, 0, 0)),
    ],
    out_specs=[
        pl.BlockSpec((BN, 128), lambda i, j: (i, 0)),
        pl.BlockSpec((BN, 128), lambda i, j: (i, 0)),
        pl.BlockSpec((BN, 2 * HEADS), lambda i, j: (i, 0)),
        pl.BlockSpec((1, 2 * HEADS), lambda i, j: (0, 0)),
    ],
    out_shape=[
        jax.ShapeDtypeStruct((N, 128), jnp.float32),
        jax.ShapeDtypeStruct((N, 128), jnp.float32),
        jax.ShapeDtypeStruct((N, 2 * HEADS), jnp.float32),
        jax.ShapeDtypeStruct((1, 2 * HEADS), jnp.float32),
    ],
)

_mesh = plsc.VectorSubcoreMesh(
    core_axis_name="c", subcore_axis_name="s", num_cores=2, num_subcores=16
)
_sc_params = pltpu.CompilerParams(
    needs_layout_passes=False, use_tc_tiling_on_sc=False
)


def _phase_b_body(aseg_hbm, src_hbm, dst_hbm, m16_hbm, z8_hbm, rec_hbm,
                  sp32_hbm, adb, asb, rec_c, srcc, dstc, m_v, s_loc, sem):
    c = lax.axis_index("c")
    s = lax.axis_index("s")
    wid = s * 2 + c
    e0 = wid * EPT_B

    pltpu.sync_copy(m16_hbm, m_v)
    pltpu.sync_copy(z8_hbm, s_loc)

    iota = lax.iota(jnp.int32, 16)
    mh = [m_v[pl.ds(h * 16, 16)] for h in range(HEADS)]

    def chunk(ch, carry):
        eb = e0 + ch * CHB
        pltpu.sync_copy(src_hbm.at[pl.ds(eb, CHB)], srcc)
        pltpu.sync_copy(dst_hbm.at[pl.ds(eb, CHB)], dstc)
        pltpu.async_copy(aseg_hbm.at[dstc], adb, sem).wait()
        pltpu.async_copy(aseg_hbm.at[srcc], asb, sem).wait()
        for g in range(CHB // 16):
            rows = g * 16 + iota
            msk = (eb + g * 16 + iota) < E
            d = dstc[pl.ds(g * 16, 16)]
            sv = srcc[pl.ds(g * 16, 16)]
            plsc.store_scatter(rec_c, [rows, jnp.full((16,), 0, jnp.int32)],
                               plsc.bitcast(sv, jnp.float32))
            plsc.store_scatter(rec_c, [rows, jnp.full((16,), 1, jnp.int32)],
                               plsc.bitcast(d, jnp.float32))
            for h in range(HEADS):
                hd = jnp.full((16,), h, jnp.int32)
                hs = jnp.full((16,), HEADS + h, jnp.int32)
                ad = plsc.load_gather(adb, [rows, hd])
                aj = plsc.load_gather(asb, [rows, hs])
                a = ad + aj
                a = jnp.where(a >= 0.0, a, jnp.float32(NEG_SLOPE) * a)
                p = jnp.exp(a - mh[h])
                p = jnp.where(msk, p, 0.0)
                plsc.store_scatter(rec_c,
                                   [rows, jnp.full((16,), 2 + h, jnp.int32)],
                                   p)
                plsc.addupdate_scatter(s_loc, [d, hd], p)
        pltpu.sync_copy(rec_c, rec_hbm.at[pl.ds(eb, CHB)])
        return carry

    lax.fori_loop(0, NCHB, chunk, 0)

    for s2 in range(16):
        pltpu.sync_copy(s_loc.at[pl.ds(NPT * s2, NPT)], sp32_hbm.at[wid, s2])


_phase_b = pl.kernel(
    _phase_b_body,
    out_type=(
        jax.ShapeDtypeStruct((E_PAD, 8), jnp.float32),
        jax.ShapeDtypeStruct((NTILES, 16, NPT, 8), jnp.float32),
    ),
    mesh=_mesh,
    compiler_params=_sc_params,
    scratch_types=[
        pltpu.VMEM((CHB, 2 * HEADS), jnp.float32),
        pltpu.VMEM((CHB, 2 * HEADS), jnp.float32),
        pltpu.VMEM((CHB, 8), jnp.float32),
        pltpu.VMEM((CHB,), jnp.int32),
        pltpu.VMEM((CHB,), jnp.int32),
        pltpu.VMEM((4 * 16,), jnp.float32),
        pltpu.VMEM((N, 8), jnp.float32),
        pltpu.SemaphoreType.DMA,
    ],
)


def _phase_c_body(h0_hbm, h1_hbm, rec_hbm, dst_hbm, sp32_hbm, bias_hbm,
                  z128_hbm, out_hbm,
                  acc, rec_sel, h_sel, srcsel, fbuf, pend, dstblk,
                  stmp, rsum, bias_v, sem, sem2):
    c = lax.axis_index("c")
    s = lax.axis_index("s")
    n0 = s * NPT

    pltpu.sync_copy(z128_hbm.at[pl.ds(n0, NPT)], acc)
    iota = lax.iota(jnp.int32, 16)

    def zf(g, carry):
        pend[pl.ds(g * 16, 16)] = jnp.zeros((16,), jnp.int32)
        return carry

    lax.fori_loop(0, PEND // 16, zf, 0)

    def main(h_hbm, cc):
        def flush(cnt, nproc):
            base = cnt - nproc if isinstance(nproc, int) else 0
            for g2 in range(FB // 16):
                ids = plsc.load_gather(pend, [base + g2 * 16 + iota])
                fbuf[pl.ds(g2 * 16, 16)] = ids
            cp1 = pltpu.async_copy(rec_hbm.at[fbuf.at[pl.ds(0, 128)]],
                                   rec_sel.at[pl.ds(0, 128)], sem)
            cp2 = pltpu.async_copy(rec_hbm.at[fbuf.at[pl.ds(128, 128)]],
                                   rec_sel.at[pl.ds(128, 128)], sem)
            cp1.wait()
            cp2.wait()
            for g2 in range(FB // 16):
                svf = plsc.load_gather(
                    rec_sel, [g2 * 16 + iota, jnp.full((16,), 0, jnp.int32)])
                srcsel[pl.ds(g2 * 16, 16)] = plsc.bitcast(svf, jnp.int32)
            cp1 = pltpu.async_copy(h_hbm.at[srcsel.at[pl.ds(0, 128)]],
                                   h_sel.at[pl.ds(0, 128)], sem2)
            cp2 = pltpu.async_copy(h_hbm.at[srcsel.at[pl.ds(128, 128)]],
                                   h_sel.at[pl.ds(128, 128)], sem2)
            cp1.wait()
            cp2.wait()

            def per_edge(e, carry2):
                er = jnp.full((16,), e, jnp.int32)
                dsp = plsc.bitcast(
                    plsc.load_gather(rec_sel,
                                     [er, jnp.full((16,), 1, jnp.int32)]),
                    jnp.int32)
                rel = dsp - n0
                w0 = plsc.load_gather(
                    rec_sel, [er, jnp.full((16,), 2 + 2 * cc, jnp.int32)])
                w1 = plsc.load_gather(
                    rec_sel, [er, jnp.full((16,), 3 + 2 * cc, jnp.int32)])
                for j8 in range(8):
                    hv = h_sel[e, pl.ds(j8 * 16, 16)]
                    w = w0 if j8 < 4 else w1
                    plsc.addupdate_scatter(acc, [rel, j8 * 16 + iota], hv * w)
                return carry2

            lax.fori_loop(0, nproc, per_edge, 0)

        def block(b, cnt):
            pltpu.sync_copy(dst_hbm.at[pl.ds(b * BLK, BLK)], dstblk)
            for g in range(BLK // 16):
                d = dstblk[pl.ds(g * 16, 16)]
                msk = jnp.logical_and(d >= n0, d < n0 + NPT)
                eid = b * BLK + g * 16 + iota
                cum = plsc.cumsum(msk.astype(jnp.int32))
                plsc.store_scatter(pend, [cnt + cum - 1], eid, mask=msk)
                cnt = cnt + cum[15]

            def wbody(cnt2):
                flush(cnt2, FB)
                return cnt2 - FB

            cnt = lax.while_loop(lambda c2: c2 >= FB, wbody, cnt)
            return cnt

        cnt = lax.fori_loop(0, E // BLK, block, jnp.int32(0))
        flush(cnt, cnt)

    @pl.when(c == 0)
    def _():
        main(h0_hbm, 0)

    @pl.when(c == 1)
    def _():
        main(h1_hbm, 1)

    def writeout(cc):
        pltpu.sync_copy(bias_hbm.at[pl.ds(cc * 128, 128)], bias_v)

        def zr(g, carry):
            rsum[pl.ds(g * 16, 16)] = jnp.zeros((16,), jnp.float32)
            return carry

        lax.fori_loop(0, (NPT * 8 + 15) // 16, zr, 0)
        for t in range(NTILES):
            pltpu.sync_copy(sp32_hbm.at[t, s], stmp.at[pl.ds(0, NPT)])

            def racc(g, carry):
                rowv = g * 2 + jnp.right_shift(iota, 3)
                colv = jnp.bitwise_and(iota, 7)
                v = plsc.load_gather(stmp, [rowv, colv])
                rsum[pl.ds(g * 16, 16)] = rsum[pl.ds(g * 16, 16)] + v
                return carry

            lax.fori_loop(0, (NPT * 8 + 15) // 16, racc, 0)

        def rinv(g, carry):
            v = rsum[pl.ds(g * 16, 16)]
            rsum[pl.ds(g * 16, 16)] = 1.0 / (v + 1e-16)
            return carry

        lax.fori_loop(0, (NPT * 8 + 15) // 16, rinv, 0)

        def row(e, carry):
            rv0 = plsc.load_gather(
                rsum, [jnp.full((16,), 0, jnp.int32) + e * 8 + 2 * cc])
            rv1 = plsc.load_gather(
                rsum, [jnp.full((16,), 0, jnp.int32) + e * 8 + 2 * cc + 1])
            for j8 in range(8):
                hv = acc[e, pl.ds(j8 * 16, 16)]
                rv = rv0 if j8 < 4 else rv1
                acc[e, pl.ds(j8 * 16, 16)] = hv * rv + bias_v[pl.ds(j8 * 16, 16)]
            return carry

        lax.fori_loop(0, NPT, row, 0)
        pltpu.sync_copy(acc, out_hbm.at[cc, pl.ds(n0, NPT)])

    @pl.when(c == 0)
    def _():
        writeout(0)

    @pl.when(c == 1)
    def _():
        writeout(1)


_phase_c = pl.kernel(
    _phase_c_body,
    out_type=jax.ShapeDtypeStruct((2, N, 128), jnp.float32),
    mesh=_mesh,
    compiler_params=_sc_params,
    scratch_types=[
        pltpu.VMEM((NPT, 128), jnp.float32),
        pltpu.VMEM((FB, 8), jnp.float32),
        pltpu.VMEM((FB, 128), jnp.float32),
        pltpu.VMEM((FB,), jnp.int32),
        pltpu.VMEM((FB,), jnp.int32),
        pltpu.VMEM((PEND,), jnp.int32),
        pltpu.VMEM((BLK,), jnp.int32),
        pltpu.VMEM((NPT + 1, 8), jnp.float32),
        pltpu.VMEM(((NPT * 8 + 15) // 16 * 16,), jnp.float32),
        pltpu.VMEM((128,), jnp.float32),
        pltpu.SemaphoreType.DMA,
        pltpu.SemaphoreType.DMA,
    ],
)


def kernel(x, edge_index, mask, weight, att, bias):
    src = edge_index[0].astype(jnp.int32)
    dst = edge_index[1].astype(jnp.int32)
    x = x.astype(jnp.float32)
    weight = weight.astype(jnp.float32)
    att = att.astype(jnp.float32)
    bias = bias.astype(jnp.float32)

    att1 = att[0]
    attd = att1[:, :OUT_CH]
    atts = att1[:, OUT_CH:]
    eye = jnp.eye(HEADS, dtype=jnp.float32)
    a_d = (eye[:, None, :] * attd[:, :, None]).reshape(HC, HEADS)
    a_s = (eye[:, None, :] * atts[:, :, None]).reshape(HC, HEADS)
    acat = jnp.concatenate([a_d, a_s], axis=1).reshape(2, 128, 2 * HEADS)

    h0, h1, aseg, m = _phase_a(x, weight, acat)

    m4 = m[0, :HEADS] + m[0, HEADS:]
    ml = jnp.where(m4 >= 0.0, m4, NEG_SLOPE * m4)
    m16 = jnp.repeat(ml, 16)

    srcp = jnp.concatenate([src, jnp.zeros((E_PAD - E,), jnp.int32)])
    dstp = jnp.concatenate([dst, jnp.zeros((E_PAD - E,), jnp.int32)])
    z8 = jnp.zeros((N, 8), jnp.float32)
    rec, sp32 = _phase_b(aseg, srcp, dstp, m16, z8)

    z128 = jnp.zeros((N, 128), jnp.float32)
    outp = _phase_c(h0, h1, rec, dst, sp32, bias, z128)
    return jnp.moveaxis(outp, 0, 1).reshape(N, HC)
